# baseline (speedup 1.0000x reference)
import jax
import jax.numpy as jnp
from jax.experimental import pallas as pl

N_ORIG = 10000
N_SUB = 32768
E = 524288
NUM_SUBG = 1024
B = 32
P = 32
HID = 128


def _segment_mean(data, seg, num):
    s = jax.ops.segment_sum(data, seg, num_segments=num)
    c = jax.ops.segment_sum(jnp.ones((data.shape[0], 1), data.dtype), seg, num_segments=num)
    return s / jnp.maximum(c, 1.0)


def _identity_kernel(x_ref, o_ref):
    o_ref[...] = x_ref[...]


def kernel(x, subgraphs_nodes_mapper, edge_index, subgraphs_batch, coarsen_adj, gin_w1, gin_b1, gin_w2, gin_b2, u_w, u_b, v_w, v_b, ext_w1, ext_b1, ext_w2, ext_b2, ext_w3, ext_b3, out_w, out_b):
    mapper = subgraphs_nodes_mapper
    src = edge_index[0]
    dst = edge_index[1]
    h = x[mapper]
    for i in range(3):
        if i > 0:
            subgraph = _segment_mean(h, subgraphs_batch, NUM_SUBG)
            fus = subgraph.reshape(B, P, HID)
            fus = jnp.matmul(coarsen_adj, fus).reshape(-1, HID)
            v_out = fus[subgraphs_batch] @ v_w[i - 1] + v_b[i - 1]
            sub = jax.nn.relu(subgraph[subgraphs_batch] + v_out)
            h = h + (sub @ u_w[i - 1] + u_b[i - 1])
            h = _segment_mean(h, mapper, N_ORIG)[mapper]
        agg = jax.ops.segment_sum(h[src], dst, num_segments=N_SUB)
        g = h + agg
        g = jax.nn.relu(g @ gin_w1[i] + gin_b1[i]) @ gin_w2[i] + gin_b2[i]
        h = h + g
    att = jax.nn.relu(h @ ext_w1 + ext_b1)
    att = jax.nn.relu(att @ ext_w2 + ext_b2)
    att = att @ ext_w3 + ext_b3
    x_att = jax.nn.sigmoid(att)
    edge_att = x_att[src] * x_att[dst]
    h = h * x_att
    subgraph_x = _segment_mean(h, subgraphs_batch, NUM_SUBG).reshape(B, P, HID)
    out = subgraph_x @ out_w + out_b
    out = pl.pallas_call(
        _identity_kernel,
        out_shape=jax.ShapeDtypeStruct(out.shape, out.dtype),
    )(out)
    return (out, edge_att)


# speedup vs baseline: 1.5725x; 1.5725x over previous
import functools

import jax
import jax.numpy as jnp
from jax import lax
from jax.experimental import pallas as pl
from jax.experimental.pallas import tpu as pltpu
from jax.experimental.pallas import tpu_sc as plsc

N_ORIG = 10000
N_SUB = 32768
E = 524288
NUM_SUBG = 1024
B = 32
P = 32
HID = 128

NC = 2
NS = 16
NT = NC * NS
EPT = E // NT
NR = 4
RSZ = N_SUB // NR
SEG_CAP = EPT + 128
CHUNK = 128
GARB = RSZ
ACC_ROWS = RSZ + 32
ZROWS = ACC_ROWS // NT
PAD_WORD = GARB << 15

_MESH = plsc.VectorSubcoreMesh(
    core_axis_name="c", subcore_axis_name="s", num_cores=NC, num_subcores=NS)


def _bucketize_body(src_hbm, dst_hbm, buckets_hbm, counts_hbm,
                    src_v, dst_v, stg, cnt_v):
    c = lax.axis_index("c")
    s = lax.axis_index("s")
    tid = c * NS + s
    base = pl.multiple_of(tid * EPT, 8)
    pltpu.sync_copy(src_hbm.at[pl.ds(base, EPT)], src_v)
    pltpu.sync_copy(dst_hbm.at[pl.ds(base, EPT)], dst_v)

    def step(k, cnts):
        d = dst_v[pl.ds(k * 16, 16)]
        sv = src_v[pl.ds(k * 16, 16)]
        rid = lax.shift_right_logical(d, 13)
        w = lax.shift_left(d & (RSZ - 1), 15) | sv
        new = []
        for r in range(NR):
            m = rid == r
            mi = m.astype(jnp.int32)
            csum = plsc.cumsum(mi)
            pos = (r * SEG_CAP + cnts[r]) + (csum - mi)
            plsc.store_scatter(stg, [pos], w, mask=m)
            new.append(cnts[r] + jnp.max(csum))
        return tuple(new)

    cnts = lax.fori_loop(0, EPT // 16, step, (0, 0, 0, 0))
    pad = jnp.full((16,), PAD_WORD, jnp.int32)
    lane = lax.iota(jnp.int32, 16)
    cnt_vec = jnp.zeros((16,), jnp.int32)
    for r in range(NR):
        for k in range(CHUNK // 16):
            plsc.store_scatter(
                stg, [(r * SEG_CAP + cnts[r] + k * 16) + lane], pad)
        pltpu.sync_copy(stg.at[pl.ds(r * SEG_CAP, SEG_CAP)], buckets_hbm.at[tid, r])
        cnt_vec = jnp.where(lane == r, jnp.full((16,), cnts[r], jnp.int32),
                            cnt_vec)
    cnt_v[...] = cnt_vec
    pltpu.sync_copy(cnt_v, counts_hbm.at[tid])


@functools.partial(
    pl.kernel,
    out_type=(jax.ShapeDtypeStruct((NT, NR, SEG_CAP), jnp.int32),
              jax.ShapeDtypeStruct((NT, 16), jnp.int32)),
    mesh=_MESH,
    compiler_params=pltpu.CompilerParams(needs_layout_passes=False),
    scratch_types=[
        pltpu.VMEM((EPT,), jnp.int32),
        pltpu.VMEM((EPT,), jnp.int32),
        pltpu.VMEM((NR * SEG_CAP,), jnp.int32),
        pltpu.VMEM((16,), jnp.int32),
    ],
)
def _bucketize(src_hbm, dst_hbm, buckets_hbm, counts_hbm, *scratch):
    _bucketize_body(src_hbm, dst_hbm, buckets_hbm, counts_hbm, *scratch)


def _agg_body(h_hbm, buckets_hbm, counts_hbm, zrows_hbm, agg_hbm,
              cw, sidx, didx, rows, cnt_v, acc):
    c = lax.axis_index("c")
    s = lax.axis_index("s")
    for p in range(NR // NC):
        r = c * (NR // NC) + p
        pltpu.sync_copy(zrows_hbm, acc.at[pl.ds(s * ZROWS, ZROWS)])
        plsc.subcore_barrier()
        for t_half in range(2):
            prod = t_half * NS + s
            pltpu.sync_copy(counts_hbm.at[prod], cnt_v)
            lane16 = lax.iota(jnp.int32, 16)
            cnt = jnp.max(jnp.where(lane16 == r, cnt_v[...], 0))
            nch = lax.shift_right_logical(cnt + (CHUNK - 1), 7)

            def chunk_step(j, _, prod=prod, r=r):
                pltpu.sync_copy(
                    buckets_hbm.at[prod, r, pl.ds(pl.multiple_of(j * CHUNK, 8), CHUNK)], cw)
                for k in range(CHUNK // 16):
                    w = cw[pl.ds(k * 16, 16)]
                    sidx[pl.ds(k * 16, 16)] = w & (N_SUB - 1)
                    didx[pl.ds(k * 16, 16)] = lax.shift_right_logical(w, 15)
                pltpu.sync_copy(h_hbm.at[sidx], rows)
                pltpu.sync_copy(rows, acc.at[didx], add=True)
                return 0

            lax.fori_loop(0, nch, chunk_step, 0)
        plsc.subcore_barrier()
        pltpu.sync_copy(
            acc.at[pl.ds(s * (RSZ // NS), RSZ // NS)],
            agg_hbm.at[pl.ds(r * RSZ + s * (RSZ // NS), RSZ // NS)])
        plsc.subcore_barrier()


@functools.partial(
    pl.kernel,
    out_type=jax.ShapeDtypeStruct((N_SUB, HID), jnp.float32),
    mesh=_MESH,
    compiler_params=pltpu.CompilerParams(needs_layout_passes=False),
    scratch_types=[
        pltpu.VMEM((CHUNK,), jnp.int32),
        pltpu.VMEM((CHUNK,), jnp.int32),
        pltpu.VMEM((CHUNK,), jnp.int32),
        pltpu.VMEM((CHUNK, HID), jnp.float32),
        pltpu.VMEM((16,), jnp.int32),
        pltpu.VMEM_SHARED((ACC_ROWS, HID), jnp.float32),
    ],
)
def _gin_agg(h_hbm, buckets_hbm, counts_hbm, zrows_hbm, agg_hbm, *scratch):
    _agg_body(h_hbm, buckets_hbm, counts_hbm, zrows_hbm, agg_hbm, *scratch)


def _segment_mean(data, seg, num):
    ssum = jax.ops.segment_sum(data, seg, num_segments=num)
    cnt = jax.ops.segment_sum(jnp.ones((data.shape[0], 1), data.dtype), seg,
                              num_segments=num)
    return ssum / jnp.maximum(cnt, 1.0)


def kernel(x, subgraphs_nodes_mapper, edge_index, subgraphs_batch, coarsen_adj, gin_w1, gin_b1, gin_w2, gin_b2, u_w, u_b, v_w, v_b, ext_w1, ext_b1, ext_w2, ext_b2, ext_w3, ext_b3, out_w, out_b):
    mapper = subgraphs_nodes_mapper
    src = edge_index[0]
    dst = edge_index[1]
    zrows = jnp.zeros((ZROWS, HID), jnp.float32)

    buckets, counts = _bucketize(src, dst)

    h = x[mapper]
    for i in range(3):
        if i > 0:
            subgraph = _segment_mean(h, subgraphs_batch, NUM_SUBG)
            fus = subgraph.reshape(B, P, HID)
            fus = jnp.matmul(coarsen_adj, fus).reshape(-1, HID)
            t_sub = jax.nn.relu(subgraph + fus @ v_w[i - 1] + v_b[i - 1])
            t_sub = t_sub @ u_w[i - 1] + u_b[i - 1]
            h = h + t_sub[subgraphs_batch]
            h = _segment_mean(h, mapper, N_ORIG)[mapper]
        agg = _gin_agg(h, buckets, counts, zrows)
        g = h + agg
        g = jax.nn.relu(g @ gin_w1[i] + gin_b1[i]) @ gin_w2[i] + gin_b2[i]
        h = h + g
    att = jax.nn.relu(h @ ext_w1 + ext_b1)
    att = jax.nn.relu(att @ ext_w2 + ext_b2)
    att = att @ ext_w3 + ext_b3
    x_att = jax.nn.sigmoid(att)
    edge_att = x_att[src] * x_att[dst]
    h = h * x_att
    subgraph_x = _segment_mean(h, subgraphs_batch, NUM_SUBG).reshape(B, P, HID)
    out = subgraph_x @ out_w + out_b
    return (out, edge_att)


# speedup vs baseline: 9.6270x; 6.1220x over previous
import functools

import jax
import jax.numpy as jnp
from jax import lax
from jax.experimental import pallas as pl
from jax.experimental.pallas import tpu as pltpu
from jax.experimental.pallas import tpu_sc as plsc

N_ORIG = 10000
N_SUB = 32768
E = 524288
NUM_SUBG = 1024
B = 32
P = 32
HID = 128

NC = 2
NS = 16
NT = NC * NS
EPT = E // NT
NR = 4
RSZ = N_SUB // NR
SEG_CAP = EPT + 128
CHUNK = 128
GARB = RSZ
ACC_ROWS = RSZ + 256
ZROWS = ACC_ROWS // NT
PAD_WORD = GARB << 15

_MESH = plsc.VectorSubcoreMesh(
    core_axis_name="c", subcore_axis_name="s", num_cores=NC, num_subcores=NS)


def _bucketize_body(src_hbm, dst_hbm, buckets_hbm, counts_hbm,
                    src_v, dst_v, stg, cnt_v):
    c = lax.axis_index("c")
    s = lax.axis_index("s")
    tid = c * NS + s
    base = pl.multiple_of(tid * EPT, 8)
    pltpu.sync_copy(src_hbm.at[pl.ds(base, EPT)], src_v)
    pltpu.sync_copy(dst_hbm.at[pl.ds(base, EPT)], dst_v)

    def step(k, cnts):
        d = dst_v[pl.ds(k * 16, 16)]
        sv = src_v[pl.ds(k * 16, 16)]
        rid = lax.shift_right_logical(d, 13)
        w = lax.shift_left(d & (RSZ - 1), 15) | sv
        new = []
        for r in range(NR):
            m = rid == r
            mi = m.astype(jnp.int32)
            csum = plsc.cumsum(mi)
            pos = (r * SEG_CAP + cnts[r]) + (csum - mi)
            plsc.store_scatter(stg, [pos], w, mask=m)
            new.append(cnts[r] + jnp.max(csum))
        return tuple(new)

    cnts = lax.fori_loop(0, EPT // 16, step, (0, 0, 0, 0))
    pad = jnp.full((16,), PAD_WORD, jnp.int32)
    lane = lax.iota(jnp.int32, 16)
    cnt_vec = jnp.zeros((16,), jnp.int32)
    for r in range(NR):
        for k in range(CHUNK // 16):
            plsc.store_scatter(
                stg, [(r * SEG_CAP + cnts[r] + k * 16) + lane], pad)
        pltpu.sync_copy(stg.at[pl.ds(r * SEG_CAP, SEG_CAP)], buckets_hbm.at[tid, r])
        cnt_vec = jnp.where(lane == r, jnp.full((16,), cnts[r], jnp.int32),
                            cnt_vec)
    cnt_v[...] = cnt_vec
    pltpu.sync_copy(cnt_v, counts_hbm.at[tid])


@functools.partial(
    pl.kernel,
    out_type=(jax.ShapeDtypeStruct((NT, NR, SEG_CAP), jnp.int32),
              jax.ShapeDtypeStruct((NT, 16), jnp.int32)),
    mesh=_MESH,
    compiler_params=pltpu.CompilerParams(needs_layout_passes=False),
    scratch_types=[
        pltpu.VMEM((EPT,), jnp.int32),
        pltpu.VMEM((EPT,), jnp.int32),
        pltpu.VMEM((NR * SEG_CAP,), jnp.int32),
        pltpu.VMEM((16,), jnp.int32),
    ],
)
def _bucketize(src_hbm, dst_hbm, buckets_hbm, counts_hbm, *scratch):
    _bucketize_body(src_hbm, dst_hbm, buckets_hbm, counts_hbm, *scratch)


def _agg_body(h_hbm, buckets_hbm, counts_hbm, zrows_hbm, agg_hbm,
              cw, sidx, didx, rows, cnt_v, acc):
    c = lax.axis_index("c")
    s = lax.axis_index("s")
    for p in range(NR // NC):
        r = c * (NR // NC) + p
        pltpu.sync_copy(zrows_hbm.at[pl.ds(0, ZROWS)],
                        acc.at[pl.ds(s * ZROWS, ZROWS)])
        plsc.subcore_barrier()
        for t_half in range(2):
            prod = t_half * NS + s
            pltpu.sync_copy(counts_hbm.at[prod], cnt_v)
            lane16 = lax.iota(jnp.int32, 16)
            cnt = jnp.max(jnp.where(lane16 == r, cnt_v[...], 0))
            nch = lax.shift_right_logical(cnt + (CHUNK - 1), 7)

            def chunk_step(j, _, prod=prod, r=r):
                pltpu.sync_copy(
                    buckets_hbm.at[prod, r, pl.ds(pl.multiple_of(j * CHUNK, 8), CHUNK)], cw)
                for k in range(CHUNK // 16):
                    w = cw[pl.ds(k * 16, 16)]
                    sidx[pl.ds(k * 16, 16)] = w & (N_SUB - 1)
                    didx[pl.ds(k * 16, 16)] = lax.shift_right_logical(w, 15)
                pltpu.sync_copy(h_hbm.at[sidx], rows)
                pltpu.sync_copy(rows, acc.at[didx], add=True)
                return 0

            lax.fori_loop(0, nch, chunk_step, 0)
        plsc.subcore_barrier()
        pltpu.sync_copy(
            acc.at[pl.ds(s * (RSZ // NS), RSZ // NS)],
            agg_hbm.at[pl.ds(r * RSZ + s * (RSZ // NS), RSZ // NS)])
        plsc.subcore_barrier()


@functools.partial(
    pl.kernel,
    out_type=jax.ShapeDtypeStruct((N_SUB, HID), jnp.float32),
    mesh=_MESH,
    compiler_params=pltpu.CompilerParams(needs_layout_passes=False),
    scratch_types=[
        pltpu.VMEM((CHUNK,), jnp.int32),
        pltpu.VMEM((CHUNK,), jnp.int32),
        pltpu.VMEM((CHUNK,), jnp.int32),
        pltpu.VMEM((CHUNK, HID), jnp.float32),
        pltpu.VMEM((16,), jnp.int32),
        pltpu.VMEM_SHARED((ACC_ROWS, HID), jnp.float32),
    ],
)
def _gin_agg(h_hbm, buckets_hbm, counts_hbm, zrows_hbm, agg_hbm, *scratch):
    _agg_body(h_hbm, buckets_hbm, counts_hbm, zrows_hbm, agg_hbm, *scratch)


RPT = N_SUB // NT
NCH = RPT // CHUNK

NSEG_SB = NUM_SUBG + 128
NSEG_MAP = 10240
NHIST = 11264


def _gather_rows_body(table_hbm, idx_hbm, out_hbm, idx_v, rows):
    c = lax.axis_index("c")
    s = lax.axis_index("s")
    tid = c * NS + s

    def chunk(j, _):
        base = pl.multiple_of(tid * RPT + j * CHUNK, 8)
        pltpu.sync_copy(idx_hbm.at[pl.ds(base, CHUNK)], idx_v)
        pltpu.sync_copy(table_hbm.at[idx_v], rows)
        pltpu.sync_copy(rows, out_hbm.at[pl.ds(base, CHUNK)])
        return 0

    lax.fori_loop(0, NCH, chunk, 0)


@functools.partial(
    pl.kernel,
    out_type=jax.ShapeDtypeStruct((N_SUB, HID), jnp.float32),
    mesh=_MESH,
    compiler_params=pltpu.CompilerParams(needs_layout_passes=False),
    scratch_types=[
        pltpu.VMEM((CHUNK,), jnp.int32),
        pltpu.VMEM((CHUNK, HID), jnp.float32),
    ],
)
def _gather_rows(table_hbm, idx_hbm, out_hbm, *scratch):
    _gather_rows_body(table_hbm, idx_hbm, out_hbm, *scratch)


def _segsum_sb_body(data_hbm, seg_hbm, zrows_hbm, out_hbm, seg_v, rows, acc):
    c = lax.axis_index("c")
    s = lax.axis_index("s")
    pltpu.sync_copy(zrows_hbm.at[pl.ds(0, NSEG_SB // NS)],
                    acc.at[pl.ds(s * (NSEG_SB // NS), NSEG_SB // NS)])
    plsc.subcore_barrier()

    def chunk(j, _):
        base = pl.multiple_of((c * NS + s) * RPT + j * CHUNK, 8)
        pltpu.sync_copy(seg_hbm.at[pl.ds(base, CHUNK)], seg_v)
        pltpu.sync_copy(data_hbm.at[pl.ds(base, CHUNK)], rows)
        pltpu.sync_copy(rows, acc.at[seg_v], add=True)
        return 0

    lax.fori_loop(0, NCH, chunk, 0)
    plsc.subcore_barrier()
    pltpu.sync_copy(acc.at[pl.ds(s * (NUM_SUBG // NS), NUM_SUBG // NS)],
                    out_hbm.at[c, pl.ds(s * (NUM_SUBG // NS), NUM_SUBG // NS)])


@functools.partial(
    pl.kernel,
    out_type=jax.ShapeDtypeStruct((NC, NUM_SUBG, HID), jnp.float32),
    mesh=_MESH,
    compiler_params=pltpu.CompilerParams(needs_layout_passes=False),
    scratch_types=[
        pltpu.VMEM((CHUNK,), jnp.int32),
        pltpu.VMEM((CHUNK, HID), jnp.float32),
        pltpu.VMEM_SHARED((NSEG_SB, HID), jnp.float32),
    ],
)
def _segsum_sb(data_hbm, seg_hbm, zrows_hbm, out_hbm, *scratch):
    _segsum_sb_body(data_hbm, seg_hbm, zrows_hbm, out_hbm, *scratch)


def _segsum_map_body(h_hbm, tsub_hbm, sb_hbm, map_hbm, zrows_hbm, out_hbm,
                     seg_v, sb_v, rows, trows, acc):
    c = lax.axis_index("c")
    s = lax.axis_index("s")
    pltpu.sync_copy(zrows_hbm.at[pl.ds(0, NSEG_MAP // NS)],
                    acc.at[pl.ds(s * (NSEG_MAP // NS), NSEG_MAP // NS)])
    plsc.subcore_barrier()

    def chunk(j, _):
        base = pl.multiple_of((c * NS + s) * RPT + j * CHUNK, 8)
        pltpu.sync_copy(sb_hbm.at[pl.ds(base, CHUNK)], sb_v)
        pltpu.sync_copy(map_hbm.at[pl.ds(base, CHUNK)], seg_v)
        pltpu.sync_copy(h_hbm.at[pl.ds(base, CHUNK)], rows)
        pltpu.sync_copy(tsub_hbm.at[sb_v], trows)

        def addrow(r, _):
            for k in range(HID // 16):
                rows.at[r][pl.ds(k * 16, 16)] = (
                    rows.at[r][pl.ds(k * 16, 16)]
                    + trows.at[r][pl.ds(k * 16, 16)])
            return 0

        lax.fori_loop(0, CHUNK, addrow, 0)
        pltpu.sync_copy(rows, acc.at[seg_v], add=True)
        return 0

    lax.fori_loop(0, NCH, chunk, 0)
    plsc.subcore_barrier()
    pltpu.sync_copy(acc.at[pl.ds(s * (NSEG_MAP // NS), NSEG_MAP // NS)],
                    out_hbm.at[c, pl.ds(s * (NSEG_MAP // NS), NSEG_MAP // NS)])


@functools.partial(
    pl.kernel,
    out_type=jax.ShapeDtypeStruct((NC, NSEG_MAP, HID), jnp.float32),
    mesh=_MESH,
    compiler_params=pltpu.CompilerParams(needs_layout_passes=False),
    scratch_types=[
        pltpu.VMEM((CHUNK,), jnp.int32),
        pltpu.VMEM((CHUNK,), jnp.int32),
        pltpu.VMEM((CHUNK, HID), jnp.float32),
        pltpu.VMEM((CHUNK, HID), jnp.float32),
        pltpu.VMEM_SHARED((NSEG_MAP, HID), jnp.float32),
    ],
)
def _segsum_map(h_hbm, tsub_hbm, sb_hbm, map_hbm, zrows_hbm, out_hbm, *scratch):
    _segsum_map_body(h_hbm, tsub_hbm, sb_hbm, map_hbm, zrows_hbm, out_hbm,
                     *scratch)


def _counts_body(sb_hbm, map_hbm, zrows_hbm, out_hbm, seg_v, ones, acc):
    c = lax.axis_index("c")
    s = lax.axis_index("s")
    pltpu.sync_copy(zrows_hbm.at[pl.ds(0, NHIST // NS), pl.ds(0, 16)],
                    acc.at[pl.ds(s * (NHIST // NS), NHIST // NS)])

    def initrow(r, _):
        ones.at[r][...] = jnp.ones((16,), jnp.float32)
        return 0

    lax.fori_loop(0, CHUNK, initrow, 0)
    plsc.subcore_barrier()

    def chunk_sb(j, _):
        base = pl.multiple_of((c * NS + s) * RPT + j * CHUNK, 8)
        pltpu.sync_copy(sb_hbm.at[pl.ds(base, CHUNK)], seg_v)
        pltpu.sync_copy(ones, acc.at[seg_v], add=True)
        return 0

    def chunk_map(j, _):
        base = pl.multiple_of((c * NS + s) * RPT + j * CHUNK, 8)
        pltpu.sync_copy(map_hbm.at[pl.ds(base, CHUNK)], seg_v)
        for k in range(CHUNK // 16):
            seg_v[pl.ds(k * 16, 16)] = seg_v[pl.ds(k * 16, 16)] + NUM_SUBG
        pltpu.sync_copy(ones, acc.at[seg_v], add=True)
        return 0

    lax.fori_loop(0, NCH, chunk_sb, 0)
    lax.fori_loop(0, NCH, chunk_map, 0)
    plsc.subcore_barrier()
    pltpu.sync_copy(acc.at[pl.ds(s * (NHIST // NS), NHIST // NS)],
                    out_hbm.at[c, pl.ds(s * (NHIST // NS), NHIST // NS)])


@functools.partial(
    pl.kernel,
    out_type=jax.ShapeDtypeStruct((NC, NHIST, 16), jnp.float32),
    mesh=_MESH,
    compiler_params=pltpu.CompilerParams(needs_layout_passes=False),
    scratch_types=[
        pltpu.VMEM((CHUNK,), jnp.int32),
        pltpu.VMEM((CHUNK, 16), jnp.float32),
        pltpu.VMEM_SHARED((NHIST, 16), jnp.float32),
    ],
)
def _counts(sb_hbm, map_hbm, zrows_hbm, out_hbm, *scratch):
    _counts_body(sb_hbm, map_hbm, zrows_hbm, out_hbm, *scratch)


def _edge_att_body(xatt_hbm, src_hbm, dst_hbm, out_hbm, xatt_v, s_v, d_v, o_v):
    c = lax.axis_index("c")
    s = lax.axis_index("s")
    tid = c * NS + s
    pltpu.sync_copy(xatt_hbm, xatt_v)
    base = pl.multiple_of(tid * EPT, 8)
    pltpu.sync_copy(src_hbm.at[pl.ds(base, EPT)], s_v)
    pltpu.sync_copy(dst_hbm.at[pl.ds(base, EPT)], d_v)

    def step(k, _):
        sv = plsc.load_gather(xatt_v, [s_v[pl.ds(k * 16, 16)]])
        dv = plsc.load_gather(xatt_v, [d_v[pl.ds(k * 16, 16)]])
        o_v[pl.ds(k * 16, 16)] = sv * dv
        return 0

    lax.fori_loop(0, EPT // 16, step, 0)
    pltpu.sync_copy(o_v, out_hbm.at[pl.ds(base, EPT)])


@functools.partial(
    pl.kernel,
    out_type=jax.ShapeDtypeStruct((E,), jnp.float32),
    mesh=_MESH,
    compiler_params=pltpu.CompilerParams(needs_layout_passes=False),
    scratch_types=[
        pltpu.VMEM((N_SUB,), jnp.float32),
        pltpu.VMEM((EPT,), jnp.int32),
        pltpu.VMEM((EPT,), jnp.int32),
        pltpu.VMEM((EPT,), jnp.float32),
    ],
)
def _edge_att(xatt_hbm, src_hbm, dst_hbm, out_hbm, *scratch):
    _edge_att_body(xatt_hbm, src_hbm, dst_hbm, out_hbm, *scratch)


def _segment_mean(data, seg, num):
    ssum = jax.ops.segment_sum(data, seg, num_segments=num)
    cnt = jax.ops.segment_sum(jnp.ones((data.shape[0], 1), data.dtype), seg,
                              num_segments=num)
    return ssum / jnp.maximum(cnt, 1.0)


def kernel(x, subgraphs_nodes_mapper, edge_index, subgraphs_batch, coarsen_adj, gin_w1, gin_b1, gin_w2, gin_b2, u_w, u_b, v_w, v_b, ext_w1, ext_b1, ext_w2, ext_b2, ext_w3, ext_b3, out_w, out_b):
    mapper = subgraphs_nodes_mapper
    src = edge_index[0]
    dst = edge_index[1]
    zrows = jnp.zeros((NSEG_MAP // NS, HID), jnp.float32)
    z16 = jnp.zeros((NHIST // NS, 16), jnp.float32)

    buckets, counts_b = _bucketize(src, dst)
    counts_b, sb_in, mp_in = lax.optimization_barrier(
        (counts_b, subgraphs_batch, mapper))
    hist = _counts(sb_in, mp_in, z16)
    cnt_all = hist[0, :, 0] + hist[1, :, 0]
    cnt_sb = jnp.maximum(cnt_all[:NUM_SUBG], 1.0)[:, None]
    cnt_map = jnp.maximum(cnt_all[NUM_SUBG:NUM_SUBG + N_ORIG], 1.0)[:, None]
    hist, x_in, mp_g = lax.optimization_barrier((hist, x, mapper))

    h = _gather_rows(x_in, mp_g)
    for i in range(3):
        if i > 0:
            ssum = _segsum_sb(h, subgraphs_batch, zrows)
            subgraph = (ssum[0] + ssum[1]) / cnt_sb
            fus = jnp.matmul(coarsen_adj, subgraph.reshape(B, P, HID))
            fus = fus.reshape(-1, HID)
            t_sub = jax.nn.relu(subgraph + fus @ v_w[i - 1] + v_b[i - 1])
            t_sub = t_sub @ u_w[i - 1] + u_b[i - 1]
            msum = _segsum_map(h, t_sub, subgraphs_batch, mapper, zrows)
            mean = (msum[0, :N_ORIG] + msum[1, :N_ORIG]) / cnt_map
            h = _gather_rows(mean, mapper)
        agg = _gin_agg(h, buckets, counts_b, zrows)
        g = h + agg
        g = jax.nn.relu(g @ gin_w1[i] + gin_b1[i]) @ gin_w2[i] + gin_b2[i]
        h = h + g
    att = jax.nn.relu(h @ ext_w1 + ext_b1)
    att = jax.nn.relu(att @ ext_w2 + ext_b2)
    att = att @ ext_w3 + ext_b3
    x_att = jax.nn.sigmoid(att)
    fsum = _segsum_sb(h * x_att, subgraphs_batch, zrows)
    xa_in, fsum = lax.optimization_barrier((x_att[:, 0], fsum))
    edge_att = _edge_att(xa_in, src, dst)[:, None]
    subgraph_x = ((fsum[0] + fsum[1]) / cnt_sb).reshape(B, P, HID)
    out = subgraph_x @ out_w + out_b
    return (out, edge_att)
